# Initial kernel scaffold; baseline (speedup 1.0000x reference)
import functools

import jax
import jax.numpy as jnp
from jax import lax
from jax.experimental import pallas as pl
from jax.experimental.pallas import tpu as pltpu
from jax.experimental.pallas import tpu_sc as plsc

_N = 10000
_D = 16
_GROUP = 128
_SUPER = 8
_K = _GROUP * _SUPER
_NCORES = 2
_NSUB = 16
_NTILES = _NCORES * _NSUB
_ROWS_PER_TILE = _N // _NSUB


def _mm1_body(x_ref, w_ref, o_ref):
    o_ref[...] = jnp.dot(x_ref[...], w_ref[...],
                         preferred_element_type=jnp.float32)


def _mid_body(p_ref, b1_ref, w2_ref, o_ref):
    h = p_ref[0] + p_ref[1] + b1_ref[...]
    h = jnp.maximum(h, 0.0)
    o_ref[...] = jnp.dot(h, w2_ref[...], preferred_element_type=jnp.float32)


def _out_body(p_ref, b2_ref, o_ref):
    o = p_ref[0] + p_ref[1] + b2_ref[...]
    mask = lax.broadcasted_iota(jnp.int32, o.shape, 1) < 7
    masked = jnp.where(mask, o, -1e30)
    m = jnp.max(masked, axis=1, keepdims=True)
    ex = jnp.where(mask, jnp.exp(o - m), 0.0)
    lse = jnp.log(jnp.sum(ex, axis=1, keepdims=True))
    o_ref[...] = o - m - lse


def _make_spmm(n_super):
    mesh = plsc.VectorSubcoreMesh(core_axis_name="c", subcore_axis_name="s")

    @functools.partial(
        pl.kernel,
        out_type=jax.ShapeDtypeStruct((_NCORES, _N, _D), jnp.float32),
        mesh=mesh,
        scratch_types=[
            pltpu.VMEM((_SUPER, _GROUP), jnp.int32),
            pltpu.VMEM((_SUPER, _GROUP), jnp.int32),
            pltpu.VMEM((_K,), jnp.float32),
            pltpu.VMEM((_K, _D), jnp.float32),
            pltpu.VMEM_SHARED((_N, _D), jnp.float32),
            pltpu.SemaphoreType.DMA,
        ],
    )
    def spmm(sup_hbm, src_hbm, dst_hbm, w_hbm, out_hbm,
             src_v, dst_v, w_v, rows_v, acc_sh, sem):
        cid = lax.axis_index("c")
        sid = lax.axis_index("s")

        zero16 = jnp.zeros((_D,), jnp.float32)

        @pl.loop(0, _ROWS_PER_TILE)
        def _(r):
            rows_v.at[r][...] = zero16

        pltpu.sync_copy(
            rows_v.at[pl.ds(0, _ROWS_PER_TILE)],
            acc_sh.at[pl.ds(sid * _ROWS_PER_TILE, _ROWS_PER_TILE)])
        plsc.subcore_barrier()

        tile = cid * _NSUB + sid
        g0 = tile * (n_super * _SUPER)

        @pl.loop(0, n_super)
        def _(s):
            gbase = g0 + s * _SUPER
            ebase = gbase * _GROUP
            pltpu.sync_copy(src_hbm.at[pl.ds(gbase, _SUPER)], src_v)
            pltpu.sync_copy(dst_hbm.at[pl.ds(gbase, _SUPER)], dst_v)
            pltpu.sync_copy(w_hbm.at[pl.ds(ebase, _K)], w_v)
            copies = [
                pltpu.async_copy(sup_hbm.at[src_v.at[j]],
                                 rows_v.at[pl.ds(j * _GROUP, _GROUP)], sem)
                for j in range(_SUPER)
            ]
            for cp in copies:
                cp.wait()

            @pl.loop(0, _K)
            def _(i):
                wspl = plsc.load_gather(
                    w_v, [jnp.full((_D,), i, jnp.int32)])
                rows_v.at[i][...] = rows_v.at[i][...] * wspl

            for j in range(_SUPER):
                pltpu.sync_copy(rows_v.at[pl.ds(j * _GROUP, _GROUP)],
                                acc_sh.at[dst_v.at[j]], add=True)

        plsc.subcore_barrier()
        pltpu.sync_copy(
            acc_sh.at[pl.ds(sid * _ROWS_PER_TILE, _ROWS_PER_TILE)],
            out_hbm.at[cid].at[pl.ds(sid * _ROWS_PER_TILE, _ROWS_PER_TILE)])

    return spmm


def kernel(x, edge_index, edge_weight, W1, b1, W2, b2):
    e = edge_weight.shape[0]
    per_tile_unit = _K * _NTILES
    e_pad = -(-e // per_tile_unit) * per_tile_unit
    n_super = e_pad // per_tile_unit
    n_groups = e_pad // _GROUP

    dst = jnp.pad(edge_index[0].astype(jnp.int32), (0, e_pad - e))
    src = jnp.pad(edge_index[1].astype(jnp.int32), (0, e_pad - e))
    dst2d = dst.reshape(n_groups, _GROUP)
    src2d = src.reshape(n_groups, _GROUP)
    w_pad = jnp.pad(edge_weight, (0, e_pad - e))

    w1p = jnp.pad(W1, ((0, 0), (0, _D - W1.shape[1])))
    b1p = jnp.pad(b1, (0, _D - b1.shape[0])).reshape(1, _D)
    w2p = jnp.pad(W2, ((0, _D - W2.shape[0]), (0, _D - W2.shape[1])))
    b2p = jnp.pad(b2, (0, _D - b2.shape[0])).reshape(1, _D)

    spmm = _make_spmm(n_super)

    support = pl.pallas_call(
        _mm1_body,
        out_shape=jax.ShapeDtypeStruct((_N, _D), jnp.float32),
    )(x, w1p)

    p1 = spmm(support, src2d, dst2d, w_pad)

    support2 = pl.pallas_call(
        _mid_body,
        out_shape=jax.ShapeDtypeStruct((_N, _D), jnp.float32),
    )(p1, b1p, w2p)

    p2 = spmm(support2, src2d, dst2d, w_pad)

    out16 = pl.pallas_call(
        _out_body,
        out_shape=jax.ShapeDtypeStruct((_N, _D), jnp.float32),
    )(p2, b2p)

    return out16[:, :7]


# speedup vs baseline: 8.5262x; 8.5262x over previous
import dataclasses
import functools

import jax
import jax.numpy as jnp
from jax import lax
from jax.experimental import pallas as pl
from jax.experimental.pallas import tpu as pltpu
from jax.experimental.pallas import tpu_sc as plsc

_N = 10000
_NPAD = 10240
_D = 16
_GROUP = 128
_SUPER = 8
_K = _GROUP * _SUPER
_NCORES = 2
_NSUB = 16
_NTILES = _NCORES * _NSUB
_ROWS_PER_TILE = _NPAD // _NSUB


def _mm1_body(x_ref, w_ref, o_ref):
    o_ref[...] = jnp.dot(x_ref[...], w_ref[...],
                         preferred_element_type=jnp.float32)


def _mid_body(p_ref, b1_ref, w2_ref, o_ref):
    h = p_ref[0] + p_ref[1] + b1_ref[...]
    h = jnp.maximum(h, 0.0)
    o_ref[...] = jnp.dot(h, w2_ref[...], preferred_element_type=jnp.float32)


def _out_body(p_ref, b2_ref, o_ref):
    o = p_ref[0] + p_ref[1] + b2_ref[...]
    mask = lax.broadcasted_iota(jnp.int32, o.shape, 1) < 7
    masked = jnp.where(mask, o, -1e30)
    m = jnp.max(masked, axis=1, keepdims=True)
    ex = jnp.where(mask, jnp.exp(o - m), 0.0)
    lse = jnp.log(jnp.sum(ex, axis=1, keepdims=True))
    o_ref[...] = o - m - lse


def _make_spmm(n_super):
    mesh = plsc.VectorSubcoreMesh(core_axis_name="c", subcore_axis_name="s",
                                  num_cores=_NCORES, num_subcores=_NSUB)
    cp = pltpu.CompilerParams()
    if "needs_layout_passes" in pltpu.CompilerParams.__dataclass_fields__:
        cp = dataclasses.replace(cp, needs_layout_passes=False,
                                 use_tc_tiling_on_sc=False)

    @functools.partial(
        pl.kernel,
        compiler_params=cp,
        out_type=jax.ShapeDtypeStruct((_NCORES, _NPAD, _D), jnp.float32),
        mesh=mesh,
        scratch_types=[
            pltpu.VMEM((_SUPER, _GROUP), jnp.int32),
            pltpu.VMEM((_SUPER, _GROUP), jnp.int32),
            pltpu.VMEM((_K,), jnp.float32),
            pltpu.VMEM((_K, _D), jnp.float32),
            pltpu.VMEM_SHARED((_NPAD, _D), jnp.float32),
            pltpu.SemaphoreType.DMA,
        ],
    )
    def spmm(sup_hbm, src_hbm, dst_hbm, w_hbm, out_hbm,
             src_v, dst_v, w_v, rows_v, acc_sh, sem):
        cid = lax.axis_index("c")
        sid = lax.axis_index("s")

        zero16 = jnp.zeros((_D,), jnp.float32)

        @pl.loop(0, _ROWS_PER_TILE)
        def _(r):
            rows_v.at[r][...] = zero16

        pltpu.sync_copy(
            rows_v.at[pl.ds(0, _ROWS_PER_TILE)],
            acc_sh.at[pl.ds(sid * _ROWS_PER_TILE, _ROWS_PER_TILE)])
        plsc.subcore_barrier()

        tile = cid * _NSUB + sid
        g0 = tile * (n_super * _SUPER)

        @pl.loop(0, n_super)
        def _(s):
            gbase = g0 + s * _SUPER
            ebase = gbase * _GROUP
            pltpu.sync_copy(src_hbm.at[pl.ds(gbase, _SUPER)], src_v)
            pltpu.sync_copy(dst_hbm.at[pl.ds(gbase, _SUPER)], dst_v)
            pltpu.sync_copy(w_hbm.at[pl.ds(ebase, _K)], w_v)
            copies = [
                pltpu.async_copy(sup_hbm.at[src_v.at[j]],
                                 rows_v.at[pl.ds(j * _GROUP, _GROUP)], sem)
                for j in range(_SUPER)
            ]
            for cp in copies:
                cp.wait()

            @pl.loop(0, _K)
            def _(i):
                wspl = plsc.load_gather(
                    w_v, [jnp.full((_D,), i, jnp.int32)])
                rows_v.at[i][...] = rows_v.at[i][...] * wspl

            for j in range(_SUPER):
                pltpu.sync_copy(rows_v.at[pl.ds(j * _GROUP, _GROUP)],
                                acc_sh.at[dst_v.at[j]], add=True)

        plsc.subcore_barrier()
        pltpu.sync_copy(
            acc_sh.at[pl.ds(sid * _ROWS_PER_TILE, _ROWS_PER_TILE)],
            out_hbm.at[cid].at[pl.ds(sid * _ROWS_PER_TILE, _ROWS_PER_TILE)])

    return spmm


def kernel(x, edge_index, edge_weight, W1, b1, W2, b2):
    e = edge_weight.shape[0]
    per_tile_unit = _K * _NTILES
    e_pad = -(-e // per_tile_unit) * per_tile_unit
    n_super = e_pad // per_tile_unit
    n_groups = e_pad // _GROUP

    dst = jnp.pad(edge_index[0].astype(jnp.int32), (0, e_pad - e))
    src = jnp.pad(edge_index[1].astype(jnp.int32), (0, e_pad - e))
    dst2d = dst.reshape(n_groups, _GROUP)
    src2d = src.reshape(n_groups, _GROUP)
    w_pad = jnp.pad(edge_weight, (0, e_pad - e))

    w1p = jnp.pad(W1, ((0, 0), (0, _D - W1.shape[1])))
    b1p = jnp.pad(b1, (0, _D - b1.shape[0])).reshape(1, _D)
    w2p = jnp.pad(W2, ((0, _D - W2.shape[0]), (0, _D - W2.shape[1])))
    b2p = jnp.pad(b2, (0, _D - b2.shape[0])).reshape(1, _D)

    spmm = _make_spmm(n_super)

    support = pl.pallas_call(
        _mm1_body,
        out_shape=jax.ShapeDtypeStruct((_N, _D), jnp.float32),
    )(x, w1p)
    support = jnp.pad(support, ((0, _NPAD - _N), (0, 0)))

    p1 = spmm(support, src2d, dst2d, w_pad)

    support2 = pl.pallas_call(
        _mid_body,
        out_shape=jax.ShapeDtypeStruct((_NPAD, _D), jnp.float32),
    )(p1, b1p, w2p)

    p2 = spmm(support2, src2d, dst2d, w_pad)

    out16 = pl.pallas_call(
        _out_body,
        out_shape=jax.ShapeDtypeStruct((_NPAD, _D), jnp.float32),
    )(p2, b2p)

    return out16[:_N, :7]


# speedup vs baseline: 15.8386x; 1.8576x over previous
import dataclasses
import functools

import jax
import jax.numpy as jnp
from jax import lax
from jax.experimental import pallas as pl
from jax.experimental.pallas import tpu as pltpu
from jax.experimental.pallas import tpu_sc as plsc

_N = 10000
_NPAD = 10240
_D = 16
_GROUP = 128
_SUPER = 8
_K = _GROUP * _SUPER
_NCORES = 2
_NSUB = 16
_NTILES = _NCORES * _NSUB
_ROWS_PER_TILE = _NPAD // _NSUB


def _mm1_body(x_ref, w_ref, o_ref):
    o_ref[...] = jnp.dot(x_ref[...], w_ref[...],
                         preferred_element_type=jnp.float32)


def _mid_body(p_ref, b1_ref, w2_ref, o_ref):
    h = p_ref[0] + p_ref[1] + b1_ref[...]
    h = jnp.maximum(h, 0.0)
    o_ref[...] = jnp.dot(h, w2_ref[...], preferred_element_type=jnp.float32)


def _out_body(p_ref, b2_ref, o_ref):
    o = p_ref[0] + p_ref[1] + b2_ref[...]
    mask = lax.broadcasted_iota(jnp.int32, o.shape, 1) < 7
    masked = jnp.where(mask, o, -1e30)
    m = jnp.max(masked, axis=1, keepdims=True)
    ex = jnp.where(mask, jnp.exp(o - m), 0.0)
    lse = jnp.log(jnp.sum(ex, axis=1, keepdims=True))
    o_ref[...] = o - m - lse


def _make_spmm(n_super):
    mesh = plsc.VectorSubcoreMesh(core_axis_name="c", subcore_axis_name="s",
                                  num_cores=_NCORES, num_subcores=_NSUB)
    cp = pltpu.CompilerParams()
    if "needs_layout_passes" in pltpu.CompilerParams.__dataclass_fields__:
        cp = dataclasses.replace(cp, needs_layout_passes=False,
                                 use_tc_tiling_on_sc=False)

    n_tile_groups = n_super * _SUPER
    n_tile_edges = n_tile_groups * _GROUP
    nbuf = 4

    @functools.partial(
        pl.kernel,
        compiler_params=cp,
        out_type=jax.ShapeDtypeStruct((_NCORES, _NPAD, _D), jnp.float32),
        mesh=mesh,
        scratch_types=[
            pltpu.VMEM((n_tile_groups, _GROUP), jnp.int32),
            pltpu.VMEM((n_tile_groups, _GROUP), jnp.int32),
            pltpu.VMEM((n_tile_edges,), jnp.float32),
            [pltpu.VMEM((_K, _D), jnp.float32) for _ in range(nbuf)],
            pltpu.VMEM_SHARED((_NPAD, _D), jnp.float32),
            [pltpu.SemaphoreType.DMA for _ in range(nbuf)],
            [pltpu.SemaphoreType.DMA for _ in range(nbuf)],
        ],
    )
    def spmm(sup_hbm, src_hbm, dst_hbm, w_hbm, out_hbm,
             src_v, dst_v, w_v, rows, acc_sh, gsem, ssem):
        cid = lax.axis_index("c")
        sid = lax.axis_index("s")
        tile = cid * _NSUB + sid
        g0 = tile * n_tile_groups

        pltpu.sync_copy(src_hbm.at[pl.ds(g0, n_tile_groups)], src_v)
        pltpu.sync_copy(dst_hbm.at[pl.ds(g0, n_tile_groups)], dst_v)
        pltpu.sync_copy(w_hbm.at[pl.ds(g0 * _GROUP, n_tile_edges)], w_v)

        def issue_gathers(s, b):
            for j in range(_SUPER):
                pltpu.async_copy(sup_hbm.at[src_v.at[s * _SUPER + j]],
                                 rows[b].at[pl.ds(j * _GROUP, _GROUP)],
                                 gsem[b])

        def wait_gathers(b):
            pltpu.make_async_copy(sup_hbm.at[pl.ds(0, _K)], rows[b],
                                  gsem[b]).wait()

        def issue_scatters(s, b):
            for j in range(_SUPER):
                pltpu.async_copy(rows[b].at[pl.ds(j * _GROUP, _GROUP)],
                                 acc_sh.at[dst_v.at[s * _SUPER + j]],
                                 ssem[b], add=True)

        def wait_scatters(b):
            pltpu.make_async_copy(rows[b], acc_sh.at[pl.ds(0, _K)],
                                  ssem[b]).wait()

        def weight(s, b):
            base = s * _K

            @plsc.parallel_loop(0, _K, unroll=8)
            def _(i):
                wspl = plsc.load_gather(
                    w_v, [jnp.full((_D,), base + i, jnp.int32)])
                rows[b].at[i][...] = rows[b].at[i][...] * wspl

        issue_gathers(1, 1)
        issue_gathers(2, 2)
        zero16 = jnp.zeros((_D,), jnp.float32)

        @pl.loop(0, _ROWS_PER_TILE)
        def _(r):
            rows[0].at[r][...] = zero16

        pltpu.sync_copy(
            rows[0].at[pl.ds(0, _ROWS_PER_TILE)],
            acc_sh.at[pl.ds(sid * _ROWS_PER_TILE, _ROWS_PER_TILE)])
        issue_gathers(0, 0)
        plsc.subcore_barrier()

        for s in range(n_super):
            b = s % nbuf
            wait_gathers(b)
            weight(s, b)
            issue_scatters(s, b)
            t = s + nbuf - 1
            if t < n_super:
                bt = t % nbuf
                if t - nbuf >= 0:
                    wait_scatters(bt)
                issue_gathers(t, bt)
        for s in range(max(0, n_super - nbuf), n_super):
            wait_scatters(s % nbuf)

        plsc.subcore_barrier()
        pltpu.sync_copy(
            acc_sh.at[pl.ds(sid * _ROWS_PER_TILE, _ROWS_PER_TILE)],
            out_hbm.at[cid].at[pl.ds(sid * _ROWS_PER_TILE, _ROWS_PER_TILE)])

    return spmm


def kernel(x, edge_index, edge_weight, W1, b1, W2, b2):
    e = edge_weight.shape[0]
    per_tile_unit = _K * _NTILES
    e_pad = -(-e // per_tile_unit) * per_tile_unit
    n_super = e_pad // per_tile_unit
    n_groups = e_pad // _GROUP

    dst = jnp.pad(edge_index[0].astype(jnp.int32), (0, e_pad - e))
    src = jnp.pad(edge_index[1].astype(jnp.int32), (0, e_pad - e))
    dst2d = dst.reshape(n_groups, _GROUP)
    src2d = src.reshape(n_groups, _GROUP)
    w_pad = jnp.pad(edge_weight, (0, e_pad - e))

    w1p = jnp.pad(W1, ((0, 0), (0, _D - W1.shape[1])))
    b1p = jnp.pad(b1, (0, _D - b1.shape[0])).reshape(1, _D)
    w2p = jnp.pad(W2, ((0, _D - W2.shape[0]), (0, _D - W2.shape[1])))
    b2p = jnp.pad(b2, (0, _D - b2.shape[0])).reshape(1, _D)

    spmm = _make_spmm(n_super)

    support = pl.pallas_call(
        _mm1_body,
        out_shape=jax.ShapeDtypeStruct((_N, _D), jnp.float32),
    )(x, w1p)
    support = jnp.pad(support, ((0, _NPAD - _N), (0, 0)))

    p1 = spmm(support, src2d, dst2d, w_pad)

    support2 = pl.pallas_call(
        _mid_body,
        out_shape=jax.ShapeDtypeStruct((_NPAD, _D), jnp.float32),
    )(p1, b1p, w2p)

    p2 = spmm(support2, src2d, dst2d, w_pad)

    out16 = pl.pallas_call(
        _out_body,
        out_shape=jax.ShapeDtypeStruct((_NPAD, _D), jnp.float32),
    )(p2, b2p)

    return out16[:_N, :7]
